# Initial kernel scaffold; baseline (speedup 1.0000x reference)
import jax
import jax.numpy as jnp
from jax.experimental import pallas as pl


def kernel(input, target):
    raise NotImplementedError("write your pallas kernel here")


# speedup vs baseline: 36.7357x; 36.7357x over previous
import functools

import jax
import jax.numpy as jnp
from jax import lax
from jax.experimental import pallas as pl
from jax.experimental.pallas import tpu as pltpu
from jax.experimental.pallas import tpu_sc as plsc

R = 128
V = 100000
L = 16
NC, NS = 2, 16
NW = NC * NS
RPW = R // NW
CE = 20000
NCH = V // CE
NVR = CE // L
NEG = -3.0e38

_mesh = plsc.VectorSubcoreMesh(core_axis_name="c", subcore_axis_name="s")


@functools.partial(
    pl.kernel,
    out_type=jax.ShapeDtypeStruct((NW, L), jnp.float32),
    mesh=_mesh,
    compiler_params=pltpu.CompilerParams(use_tc_tiling_on_sc=False,
                                          needs_layout_passes=False),
    scratch_types=[
        pltpu.VMEM((2, CE), jnp.float32),
        pltpu.VMEM((R,), jnp.int32),
        pltpu.VMEM((L,), jnp.float32),
        pltpu.VMEM((L,), jnp.float32),
        pltpu.SemaphoreType.DMA,
        pltpu.SemaphoreType.DMA,
    ],
)
def _dlr_topk(x_hbm, tgt_hbm, out_hbm, buf, tgt_v, tv_v, loss_v, sem0, sem1):
    wid = lax.axis_index("s") * NC + lax.axis_index("c")
    base = wid * RPW
    sems = (sem0, sem1)

    pltpu.sync_copy(tgt_hbm, tgt_v)

    lanes = lax.iota(jnp.int32, L)
    lossvec = jnp.zeros((L,), jnp.float32)

    total = RPW * NCH
    cps = {0: pltpu.async_copy(x_hbm.at[base, pl.ds(0, CE)], buf.at[0], sems[0])}

    m1 = m2 = m3 = None
    for k in range(total):
        r, ch = divmod(k, NCH)
        row = base + r
        if k + 1 < total:
            rn, chn = divmod(k + 1, NCH)
            cps[k + 1] = pltpu.async_copy(
                x_hbm.at[base + rn, pl.ds(chn * CE, CE)],
                buf.at[(k + 1) % 2], sems[(k + 1) % 2])
        cps[k].wait()
        b = k % 2

        if ch == 0:
            m1 = jnp.full((L,), NEG, jnp.float32)
            m2 = jnp.full((L,), NEG, jnp.float32)
            m3 = jnp.full((L,), NEG, jnp.float32)

        def body(i, carry, _b=b):
            a1, a2, a3 = carry
            v = buf[_b, pl.ds(i * L, L)]
            gt1 = v > a1
            gt2 = v > a2
            gt3 = v > a3
            n1 = jnp.where(gt1, v, a1)
            n2 = jnp.where(gt1, a1, jnp.where(gt2, v, a2))
            n3 = jnp.where(gt2, a2, jnp.where(gt3, v, a3))
            return n1, n2, n3

        m1, m2, m3 = lax.fori_loop(0, NVR, body, (m1, m2, m3))

        if ch == NCH - 1:
            t1 = jnp.max(m1)
            sel = lanes == plsc.all_reduce_ffs(m1 == t1)
            m1p = jnp.where(sel, m2, m1)
            m2p = jnp.where(sel, m3, m2)
            t2 = jnp.max(m1p)
            sel2 = lanes == plsc.all_reduce_ffs(m1p == t2)
            m1q = jnp.where(sel2, m2p, m1p)
            t3 = jnp.max(m1q)

            blk = (base // L) * L
            yv = tgt_v[pl.ds(blk, L)]
            y = jnp.max(jnp.where(lanes == row - blk, yv, 0))
            st = (y // L) * L
            pltpu.sync_copy(x_hbm.at[row, pl.ds(st, L)], tv_v)
            xv = jnp.max(jnp.where(lanes == y - st, tv_v[...], NEG))

            num = jnp.where(xv == t1, t2 - xv, t1 - xv)
            den = t1 - t3 + 1e-12
            ratio = jnp.broadcast_to(num, (L,)) / jnp.broadcast_to(den, (L,))
            lossvec = jnp.where(lanes == r, ratio, lossvec)

    loss_v[...] = lossvec
    pltpu.sync_copy(loss_v, out_hbm.at[wid])


def kernel(input, target):
    out = _dlr_topk(input, target)
    return jnp.sum(out) / R
